# Initial kernel scaffold; baseline (speedup 1.0000x reference)
import jax
import jax.numpy as jnp
from jax.experimental import pallas as pl


def kernel(x, edge_index, edge_attr, W1, b1, W2, b2, Wih_f, Whh_f, bih_f, bhh_f, Wih_b, Whh_b, bih_b, bhh_b, Wl, bl):
    raise NotImplementedError("write your pallas kernel here")


# speedup vs baseline: 12.7427x; 12.7427x over previous
import functools

import jax
import jax.numpy as jnp
from jax import lax
from jax.experimental import pallas as pl
from jax.experimental.pallas import tpu as pltpu
from jax.experimental.pallas import tpu_sc as plsc

NN = 10000
NE = 320000
D = 128
LH = 64
G4 = 4 * LH

NC = 2
NS = 16
L = 16
NW = NC * NS
CHUNK = 128
EPW = -(-NE // NW)
EPW_PAD = -(-EPW // CHUNK) * CHUNK
NCHUNK = EPW_PAD // CHUNK
NNP = 10240
NPS = 624
NREM = NN - NS * NPS

def _sc_deg_body(col_hbm, ea_hbm, pdeg_hbm, colv, eav, degv):
    cid = lax.axis_index("c")
    sid = lax.axis_index("s")
    wid = sid * NC + cid
    pltpu.sync_copy(col_hbm.at[wid], colv)
    pltpu.sync_copy(ea_hbm.at[wid], eav)

    def zero(i, _):
        degv[pl.ds(i * L, L)] = jnp.zeros((L,), jnp.float32)
        return 0

    lax.fori_loop(0, NNP // L, zero, 0)

    NPC = CHUNK // L

    def body(i, _):
        r = i // NPC
        q = i % NPC
        idx = colv[r, pl.ds(q * L, L)]
        vals = eav[r, pl.ds(q * L, L)]
        plsc.addupdate_scatter(degv, [idx], vals)
        return 0

    lax.fori_loop(0, NCHUNK * NPC, body, 0)
    pltpu.sync_copy(degv, pdeg_hbm.at[pl.ds(wid * NNP, NNP)])


def _sc_agg_body(y_hbm, row_hbm, col_hbm, ea_hbm, out_hbm, rowv, colv, eav,
                 bufv, aggs, sem):
    cid = lax.axis_index("c")
    sid = lax.axis_index("s")
    wid = sid * NC + cid
    pltpu.sync_copy(row_hbm.at[wid], rowv)
    pltpu.sync_copy(col_hbm.at[wid], colv)
    pltpu.sync_copy(ea_hbm.at[wid], eav)

    def zbuf(i, _):
        r = i // (D // L)
        q = i % (D // L)
        bufv[r, pl.ds(q * L, L)] = jnp.zeros((L,), jnp.float32)
        return 0

    lax.fori_loop(0, CHUNK * (D // L), zbuf, 0)

    def zagg(k, _):
        pltpu.sync_copy(bufv.at[pl.ds(0, CHUNK)],
                        aggs.at[pl.ds(sid * NPS + k * CHUNK, CHUNK)])
        return 0

    lax.fori_loop(0, 4, zagg, 0)
    pltpu.sync_copy(bufv.at[pl.ds(0, NPS - 4 * CHUNK)],
                    aggs.at[pl.ds(sid * NPS + 4 * CHUNK, NPS - 4 * CHUNK)])

    @pl.when(sid == NS - 1)
    def _():
        pltpu.sync_copy(bufv.at[pl.ds(0, NREM)],
                        aggs.at[pl.ds(NS * NPS, NREM)])

    plsc.subcore_barrier()

    def chunk_body(j, _):
        pltpu.async_copy(y_hbm.at[rowv.at[j]], bufv, sem).wait()

        def sgroup(g0, _):
            ev = eav[j, pl.ds(g0 * L, L)]
            for t in range(L):
                sv = jnp.full((L,), ev[t], jnp.float32)
                r = g0 * L + t
                for q in range(D // L):
                    bufv[r, pl.ds(q * L, L)] = bufv[r, pl.ds(q * L, L)] * sv
            return 0

        lax.fori_loop(0, CHUNK // L, sgroup, 0)
        pltpu.sync_copy(bufv, aggs.at[colv.at[j]], add=True)
        return 0

    lax.fori_loop(0, NCHUNK, chunk_body, 0)
    plsc.subcore_barrier()
    pltpu.sync_copy(aggs.at[pl.ds(sid * NPS, NPS)],
                    out_hbm.at[pl.ds(cid * NN + sid * NPS, NPS)])

    @pl.when(sid == NS - 1)
    def _():
        pltpu.sync_copy(aggs.at[pl.ds(NS * NPS, NREM)],
                        out_hbm.at[pl.ds(cid * NN + NS * NPS, NREM)])


def _tc_dinv_body(pdeg_ref, dinv_ref):
    deg = jnp.sum(pdeg_ref[...], axis=0, keepdims=True)
    safe = jnp.where(deg > 0, deg, 1.0)
    dinv_ref[...] = jnp.where(deg > 0, lax.rsqrt(safe), 0.0)


_tc_dinv = pl.pallas_call(
    _tc_dinv_body, out_shape=jax.ShapeDtypeStruct((1, NNP), jnp.float32))


def _tc_y1_body(x_ref, dinv_ref, w_ref, y_ref):
    y_ref[...] = jnp.dot(x_ref[...] * dinv_ref[...], w_ref[...],
                         preferred_element_type=jnp.float32)


_tc_y1 = pl.pallas_call(
    _tc_y1_body, out_shape=jax.ShapeDtypeStruct((NN, D), jnp.float32))


def _tc_mid_body(agg_ref, dinv_ref, b1_ref, w2_ref, y2_ref):
    h1 = jax.nn.relu(
        (agg_ref[0:NN, :] + agg_ref[NN:2 * NN, :]) * dinv_ref[...] + b1_ref[...])
    y2_ref[...] = jnp.dot(h1 * dinv_ref[...], w2_ref[...],
                          preferred_element_type=jnp.float32)


_tc_mid = pl.pallas_call(
    _tc_mid_body, out_shape=jax.ShapeDtypeStruct((NN, D), jnp.float32))


def _tc_lstm_body(agg_ref, dinv_ref, b2_ref, wihtf_ref, wihtb_ref, whhtf_ref,
                  whhtb_ref, biasf_ref, biasb_ref, wl_ref, bl_ref, out_ref,
                  ginf, ginb, hcat):
    h2 = jax.nn.relu(
        (agg_ref[0:NN, :] + agg_ref[NN:2 * NN, :]) * dinv_ref[...] + b2_ref[...])
    ginf[...] = jnp.dot(h2, wihtf_ref[...],
                        preferred_element_type=jnp.float32) + biasf_ref[...]
    ginb[...] = jnp.dot(h2, wihtb_ref[...],
                        preferred_element_type=jnp.float32) + biasb_ref[...]

    def step(t, carry):
        hf, cf, hb, cb = carry
        gf = ginf[pl.ds(t, 1), :] + jnp.dot(hf, whhtf_ref[...],
                                            preferred_element_type=jnp.float32)
        i_ = jax.nn.sigmoid(gf[:, 0:LH])
        f_ = jax.nn.sigmoid(gf[:, LH:2 * LH])
        g_ = jnp.tanh(gf[:, 2 * LH:3 * LH])
        o_ = jax.nn.sigmoid(gf[:, 3 * LH:4 * LH])
        cf = f_ * cf + i_ * g_
        hf = o_ * jnp.tanh(cf)
        hcat[pl.ds(t, 1), 0:LH] = hf

        rt = NN - 1 - t
        gb = ginb[pl.ds(rt, 1), :] + jnp.dot(hb, whhtb_ref[...],
                                             preferred_element_type=jnp.float32)
        ib = jax.nn.sigmoid(gb[:, 0:LH])
        fb = jax.nn.sigmoid(gb[:, LH:2 * LH])
        gb_ = jnp.tanh(gb[:, 2 * LH:3 * LH])
        ob = jax.nn.sigmoid(gb[:, 3 * LH:4 * LH])
        cb = fb * cb + ib * gb_
        hb = ob * jnp.tanh(cb)
        hcat[pl.ds(rt, 1), LH:2 * LH] = hb
        return hf, cf, hb, cb

    z = jnp.zeros((1, LH), jnp.float32)
    lax.fori_loop(0, NN, step, (z, z, z, z))
    p = jnp.dot(hcat[...], wl_ref[...],
                preferred_element_type=jnp.float32) + bl_ref[...]
    out_ref[...] = jax.nn.sigmoid(p)


_tc_lstm = pl.pallas_call(
    _tc_lstm_body,
    out_shape=jax.ShapeDtypeStruct((NN, 1), jnp.float32),
    scratch_shapes=[
        pltpu.VMEM((NN, G4), jnp.float32),
        pltpu.VMEM((NN, G4), jnp.float32),
        pltpu.VMEM((NN, 2 * LH), jnp.float32),
    ],
)


@functools.lru_cache(maxsize=1)
def _sc_kernels():
    mesh = plsc.VectorSubcoreMesh(core_axis_name="c", subcore_axis_name="s",
                                  num_cores=NC, num_subcores=NS)
    params = pltpu.CompilerParams(needs_layout_passes=False)
    sc_deg = pl.kernel(
        _sc_deg_body,
        out_type=jax.ShapeDtypeStruct((NW * NNP,), jnp.float32),
        mesh=mesh,
        compiler_params=params,
        scratch_types=[
            pltpu.VMEM((NCHUNK, CHUNK), jnp.int32),
            pltpu.VMEM((NCHUNK, CHUNK), jnp.float32),
            pltpu.VMEM((NNP,), jnp.float32),
        ],
    )
    sc_agg = pl.kernel(
        _sc_agg_body,
        out_type=jax.ShapeDtypeStruct((NC * NN, D), jnp.float32),
        mesh=mesh,
        compiler_params=params,
        scratch_types=[
            pltpu.VMEM((NCHUNK, CHUNK), jnp.int32),
            pltpu.VMEM((NCHUNK, CHUNK), jnp.int32),
            pltpu.VMEM((NCHUNK, CHUNK), jnp.float32),
            pltpu.VMEM((CHUNK, D), jnp.float32),
            pltpu.VMEM_SHARED((NN, D), jnp.float32),
            pltpu.SemaphoreType.DMA,
        ],
    )
    return sc_deg, sc_agg


def kernel(x, edge_index, edge_attr, W1, b1, W2, b2, Wih_f, Whh_f, bih_f,
           bhh_f, Wih_b, Whh_b, bih_b, bhh_b, Wl, bl):
    _sc_deg, _sc_agg = _sc_kernels()
    row = edge_index[0]
    col = edge_index[1]
    pad = NW * EPW_PAD - NE
    zi = jnp.zeros((pad,), jnp.int32)
    rowp = jnp.concatenate([row, zi]).reshape(NW, NCHUNK, CHUNK)
    colp3 = jnp.concatenate([col, zi]).reshape(NW, NCHUNK, CHUNK)
    eap = jnp.concatenate([edge_attr, jnp.zeros((pad,), jnp.float32)])
    eap = eap.reshape(NW, NCHUNK, CHUNK)

    pdeg = _sc_deg(colp3, eap).reshape(NW, NNP)
    dinv = _tc_dinv(pdeg)[:, :NN].reshape(NN, 1)

    y1 = _tc_y1(x, dinv, W1)
    agg1 = _sc_agg(y1, rowp, colp3, eap)
    y2 = _tc_mid(agg1, dinv, b1.reshape(1, D), W2)
    agg2 = _sc_agg(y2, rowp, colp3, eap)

    out = _tc_lstm(agg2, dinv, b2.reshape(1, D), Wih_f.T, Wih_b.T, Whh_f.T,
                   Whh_b.T, (bih_f + bhh_f).reshape(1, G4),
                   (bih_b + bhh_b).reshape(1, G4), Wl, bl.reshape(1, 1))
    return out.reshape(1, NN)


# speedup vs baseline: 22.5135x; 1.7668x over previous
import functools

import jax
import jax.numpy as jnp
from jax import lax
from jax.experimental import pallas as pl
from jax.experimental.pallas import tpu as pltpu
from jax.experimental.pallas import tpu_sc as plsc

NN = 10000
NE = 320000
D = 128
LH = 64
G4 = 4 * LH

NC = 2
NS = 16
L = 16
NW = NC * NS
CHUNK = 128
EPW = -(-NE // NW)
EPW_PAD = -(-EPW // CHUNK) * CHUNK
NCHUNK = EPW_PAD // CHUNK
NNP = 10240
NPS = 624
NREM = NN - NS * NPS

def _sc_deg_body(col_hbm, ea_hbm, pdeg_hbm, colv, eav, degv):
    cid = lax.axis_index("c")
    sid = lax.axis_index("s")
    wid = sid * NC + cid
    pltpu.sync_copy(col_hbm.at[wid], colv)
    pltpu.sync_copy(ea_hbm.at[wid], eav)

    def zero(i, _):
        degv[pl.ds(i * L, L)] = jnp.zeros((L,), jnp.float32)
        return 0

    lax.fori_loop(0, NNP // L, zero, 0)

    NPC = CHUNK // L

    def body(i, _):
        r = i // NPC
        q = i % NPC
        idx = colv[r, pl.ds(q * L, L)]
        vals = eav[r, pl.ds(q * L, L)]
        plsc.addupdate_scatter(degv, [idx], vals)
        return 0

    lax.fori_loop(0, NCHUNK * NPC, body, 0)
    pltpu.sync_copy(degv, pdeg_hbm.at[pl.ds(wid * NNP, NNP)])


def _sc_agg_body(y_hbm, row_hbm, col_hbm, ea_hbm, out_hbm, rowv, colv, eav,
                 bufv, aggs, sem):
    cid = lax.axis_index("c")
    sid = lax.axis_index("s")
    wid = sid * NC + cid
    pltpu.sync_copy(row_hbm.at[wid], rowv)
    pltpu.sync_copy(col_hbm.at[wid], colv)
    pltpu.sync_copy(ea_hbm.at[wid], eav)

    def zbuf(i, _):
        r = i // (D // L)
        q = i % (D // L)
        bufv[r, pl.ds(q * L, L)] = jnp.zeros((L,), jnp.float32)
        return 0

    lax.fori_loop(0, CHUNK * (D // L), zbuf, 0)

    def zagg(k, _):
        pltpu.sync_copy(bufv.at[pl.ds(0, CHUNK)],
                        aggs.at[pl.ds(sid * NPS + k * CHUNK, CHUNK)])
        return 0

    lax.fori_loop(0, 4, zagg, 0)
    pltpu.sync_copy(bufv.at[pl.ds(0, NPS - 4 * CHUNK)],
                    aggs.at[pl.ds(sid * NPS + 4 * CHUNK, NPS - 4 * CHUNK)])

    @pl.when(sid == NS - 1)
    def _():
        pltpu.sync_copy(bufv.at[pl.ds(0, NREM)],
                        aggs.at[pl.ds(NS * NPS, NREM)])

    plsc.subcore_barrier()

    def chunk_body(j, _):
        pltpu.async_copy(y_hbm.at[rowv.at[j]], bufv, sem).wait()

        def sgroup(g0, _):
            ev = eav[j, pl.ds(g0 * L, L)]
            for t in range(L):
                sv = jnp.full((L,), ev[t], jnp.float32)
                r = g0 * L + t
                for q in range(D // L):
                    bufv[r, pl.ds(q * L, L)] = bufv[r, pl.ds(q * L, L)] * sv
            return 0

        lax.fori_loop(0, CHUNK // L, sgroup, 0)
        pltpu.sync_copy(bufv, aggs.at[colv.at[j]], add=True)
        return 0

    lax.fori_loop(0, NCHUNK, chunk_body, 0)
    plsc.subcore_barrier()
    pltpu.sync_copy(aggs.at[pl.ds(sid * NPS, NPS)],
                    out_hbm.at[pl.ds(cid * NN + sid * NPS, NPS)])

    @pl.when(sid == NS - 1)
    def _():
        pltpu.sync_copy(aggs.at[pl.ds(NS * NPS, NREM)],
                        out_hbm.at[pl.ds(cid * NN + NS * NPS, NREM)])


def _tc_dinv_body(pdeg_ref, dinv_ref):
    deg = jnp.sum(pdeg_ref[...], axis=0, keepdims=True)
    safe = jnp.where(deg > 0, deg, 1.0)
    dinv_ref[...] = jnp.where(deg > 0, lax.rsqrt(safe), 0.0)


_tc_dinv = pl.pallas_call(
    _tc_dinv_body, out_shape=jax.ShapeDtypeStruct((1, NNP), jnp.float32))


def _tc_y1_body(x_ref, dinv_ref, w_ref, y_ref):
    y_ref[...] = jnp.dot(x_ref[...] * dinv_ref[...], w_ref[...],
                         preferred_element_type=jnp.float32)


_tc_y1 = pl.pallas_call(
    _tc_y1_body, out_shape=jax.ShapeDtypeStruct((NN, D), jnp.float32))


def _tc_mid_body(agg_ref, dinv_ref, b1_ref, w2_ref, y2_ref):
    h1 = jax.nn.relu(
        (agg_ref[0:NN, :] + agg_ref[NN:2 * NN, :]) * dinv_ref[...] + b1_ref[...])
    y2_ref[...] = jnp.dot(h1 * dinv_ref[...], w2_ref[...],
                          preferred_element_type=jnp.float32)


_tc_mid = pl.pallas_call(
    _tc_mid_body, out_shape=jax.ShapeDtypeStruct((NN, D), jnp.float32))


def _tc_h2_body(agg_ref, dinv_ref, b2_ref, h2_ref):
    h2_ref[...] = jax.nn.relu(
        (agg_ref[0:NN, :] + agg_ref[NN:2 * NN, :]) * dinv_ref[...] + b2_ref[...])


_tc_h2 = pl.pallas_call(
    _tc_h2_body, out_shape=jax.ShapeDtypeStruct((NN, D), jnp.float32))


def _tc_lstm_body(h2_ref, wf_ref, wb_ref, wr_ref, bc_ref,
                  wl_ref, bl_ref, out_ref, gf_s, gb_s, hcat):
    h2 = h2_ref[...]
    gf_s[...] = jnp.dot(h2, wf_ref[...],
                        preferred_element_type=jnp.float32) + bc_ref[...]
    gb_s[...] = jnp.dot(h2, wb_ref[...], preferred_element_type=jnp.float32)

    def step(t, carry):
        h, c = carry
        g = (gf_s[pl.ds(t, 1), :] + gb_s[pl.ds(NN - 1 - t, 1), :] +
             jnp.dot(h, wr_ref[...], preferred_element_type=jnp.float32))
        sg = jax.nn.sigmoid(g[:, 0:384])
        th = jnp.tanh(g[:, 384:512])
        c = sg[:, 128:256] * c + sg[:, 0:128] * th
        h = sg[:, 256:384] * jnp.tanh(c)
        hcat[pl.ds(t, 1), 0:LH] = h[:, 0:LH]
        hcat[pl.ds(NN - 1 - t, 1), LH:2 * LH] = h[:, LH:2 * LH]
        return h, c

    z = jnp.zeros((1, 2 * LH), jnp.float32)
    lax.fori_loop(0, NN, step, (z, z))
    p = jnp.dot(hcat[...], wl_ref[...],
                preferred_element_type=jnp.float32) + bl_ref[...]
    out_ref[...] = jax.nn.sigmoid(p)


_tc_lstm = pl.pallas_call(
    _tc_lstm_body,
    out_shape=jax.ShapeDtypeStruct((NN, 1), jnp.float32),
    scratch_shapes=[
        pltpu.VMEM((NN, 512), jnp.float32),
        pltpu.VMEM((NN, 512), jnp.float32),
        pltpu.VMEM((NN, 2 * LH), jnp.float32),
    ],
    compiler_params=pltpu.CompilerParams(vmem_limit_bytes=100 * 1024 * 1024),
)


def _lstm_weights(Wih_f, Whh_f, bih_f, bhh_f, Wih_b, Whh_b, bih_b, bhh_b):
    def slots(WT):
        return WT[:, 0:LH], WT[:, LH:2 * LH], WT[:, 3 * LH:4 * LH], WT[:, 2 * LH:3 * LH]

    def place(WT, off):
        i_, f_, o_, g_ = slots(WT)
        out = jnp.zeros((WT.shape[0], 512), WT.dtype)
        out = out.at[:, 0 + off:LH + off].set(i_)
        out = out.at[:, 128 + off:128 + LH + off].set(f_)
        out = out.at[:, 256 + off:256 + LH + off].set(o_)
        out = out.at[:, 384 + off:384 + LH + off].set(g_)
        return out

    WF = place(Wih_f.T, 0)
    WB = place(Wih_b.T, LH)
    WR = jnp.zeros((2 * LH, 512), Wih_f.dtype)
    WR = WR + place(jnp.concatenate([Whh_f.T, jnp.zeros_like(Whh_f.T)], 0), 0)
    WR = WR + place(jnp.concatenate([jnp.zeros_like(Whh_b.T), Whh_b.T], 0), LH)
    bc = (place((bih_f + bhh_f).reshape(1, G4), 0) +
          place((bih_b + bhh_b).reshape(1, G4), LH))
    return WF, WB, WR, bc


@functools.lru_cache(maxsize=1)
def _sc_kernels():
    mesh = plsc.VectorSubcoreMesh(core_axis_name="c", subcore_axis_name="s",
                                  num_cores=NC, num_subcores=NS)
    params = pltpu.CompilerParams(needs_layout_passes=False)
    sc_deg = pl.kernel(
        _sc_deg_body,
        out_type=jax.ShapeDtypeStruct((NW * NNP,), jnp.float32),
        mesh=mesh,
        compiler_params=params,
        scratch_types=[
            pltpu.VMEM((NCHUNK, CHUNK), jnp.int32),
            pltpu.VMEM((NCHUNK, CHUNK), jnp.float32),
            pltpu.VMEM((NNP,), jnp.float32),
        ],
    )
    sc_agg = pl.kernel(
        _sc_agg_body,
        out_type=jax.ShapeDtypeStruct((NC * NN, D), jnp.float32),
        mesh=mesh,
        compiler_params=params,
        scratch_types=[
            pltpu.VMEM((NCHUNK, CHUNK), jnp.int32),
            pltpu.VMEM((NCHUNK, CHUNK), jnp.int32),
            pltpu.VMEM((NCHUNK, CHUNK), jnp.float32),
            pltpu.VMEM((CHUNK, D), jnp.float32),
            pltpu.VMEM_SHARED((NN, D), jnp.float32),
            pltpu.SemaphoreType.DMA,
        ],
    )
    return sc_deg, sc_agg


def kernel(x, edge_index, edge_attr, W1, b1, W2, b2, Wih_f, Whh_f, bih_f,
           bhh_f, Wih_b, Whh_b, bih_b, bhh_b, Wl, bl):
    _sc_deg, _sc_agg = _sc_kernels()
    row = edge_index[0]
    col = edge_index[1]
    pad = NW * EPW_PAD - NE
    zi = jnp.zeros((pad,), jnp.int32)
    rowp = jnp.concatenate([row, zi]).reshape(NW, NCHUNK, CHUNK)
    colp3 = jnp.concatenate([col, zi]).reshape(NW, NCHUNK, CHUNK)
    eap = jnp.concatenate([edge_attr, jnp.zeros((pad,), jnp.float32)])
    eap = eap.reshape(NW, NCHUNK, CHUNK)

    pdeg = _sc_deg(colp3, eap).reshape(NW, NNP)
    dinv = _tc_dinv(pdeg)[:, :NN].reshape(NN, 1)

    y1 = _tc_y1(x, dinv, W1)
    agg1 = _sc_agg(y1, rowp, colp3, eap)
    y2 = _tc_mid(agg1, dinv, b1.reshape(1, D), W2)
    agg2 = _sc_agg(y2, rowp, colp3, eap)

    WF, WB, WR, bc = _lstm_weights(Wih_f, Whh_f, bih_f, bhh_f, Wih_b, Whh_b,
                                   bih_b, bhh_b)
    h2 = _tc_h2(agg2, dinv, b2.reshape(1, D))
    out = _tc_lstm(h2, WF, WB, WR, bc, Wl, bl.reshape(1, 1))
    return out.reshape(1, NN)
